# Initial kernel scaffold; baseline (speedup 1.0000x reference)
import jax
import jax.numpy as jnp
from jax.experimental import pallas as pl


def kernel(x, table):
    raise NotImplementedError("write your pallas kernel here")


# speedup vs baseline: 1.5662x; 1.5662x over previous
import functools

import jax
import jax.numpy as jnp
from jax import lax
from jax.experimental import pallas as pl
from jax.experimental.pallas import tpu as pltpu
from jax.experimental.pallas import tpu_sc as plsc

NUM_EMBEDDINGS = 1000000
EMBEDDING_DIM = 32
BATCH = 16384
FIELDS = 26

NC, NS = 2, 16
NW = NC * NS
B_TOT = BATCH * FIELDS
BPW = B_TOT // NW
SUB = 128
NSUB = 13
CH = SUB * NSUB
NCH = BPW // CH
assert CH * NCH == BPW


@functools.partial(
    pl.kernel,
    out_type=jax.ShapeDtypeStruct((B_TOT, EMBEDDING_DIM), jnp.float32),
    mesh=plsc.VectorSubcoreMesh(core_axis_name="c", subcore_axis_name="s"),
    compiler_params=pltpu.CompilerParams(use_tc_tiling_on_sc=False),
    scratch_types=[
        pltpu.VMEM((BPW,), jnp.int32),
        pltpu.VMEM((CH, EMBEDDING_DIM), jnp.float32),
        pltpu.SemaphoreType.DMA,
    ],
)
def _emb_lookup(x_hbm, table_hbm, out_hbm, idx_v, rows_v, sem):
    wid = lax.axis_index("s") * NC + lax.axis_index("c")
    base = wid * BPW
    pltpu.sync_copy(x_hbm.at[pl.ds(base, BPW)], idx_v)

    def chunk_body(ch, carry):
        descs = []
        for j in range(NSUB):
            d = pltpu.async_copy(
                table_hbm.at[idx_v.at[pl.ds(ch * CH + j * SUB, SUB)]],
                rows_v.at[pl.ds(j * SUB, SUB)],
                sem,
            )
            descs.append(d)
        for d in descs:
            d.wait()
        pltpu.sync_copy(rows_v, out_hbm.at[pl.ds(base + ch * CH, CH)])
        return carry

    lax.fori_loop(0, NCH, chunk_body, 0)


def kernel(x, table):
    flat = _emb_lookup(x.reshape(-1), table)
    return flat.reshape(BATCH, FIELDS, EMBEDDING_DIM)


# speedup vs baseline: 1.5827x; 1.0106x over previous
import functools

import jax
import jax.numpy as jnp
from jax import lax
from jax.experimental import pallas as pl
from jax.experimental.pallas import tpu as pltpu
from jax.experimental.pallas import tpu_sc as plsc

NUM_EMBEDDINGS = 1000000
EMBEDDING_DIM = 32
BATCH = 16384
FIELDS = 26

NC, NS = 2, 16
NW = NC * NS
B_TOT = BATCH * FIELDS
BPW = B_TOT // NW
CH = 1664
NCH = BPW // CH
assert CH * NCH == BPW


@functools.partial(
    pl.kernel,
    out_type=jax.ShapeDtypeStruct((B_TOT, EMBEDDING_DIM), jnp.float32),
    mesh=plsc.VectorSubcoreMesh(core_axis_name="c", subcore_axis_name="s"),
    compiler_params=pltpu.CompilerParams(use_tc_tiling_on_sc=False),
    scratch_types=[
        pltpu.VMEM((BPW,), jnp.int32),
        pltpu.VMEM((CH, EMBEDDING_DIM), jnp.float32),
        pltpu.VMEM((CH, EMBEDDING_DIM), jnp.float32),
        pltpu.SemaphoreType.DMA,
        pltpu.SemaphoreType.DMA,
        pltpu.SemaphoreType.DMA,
        pltpu.SemaphoreType.DMA,
    ],
)
def _emb_lookup(x_hbm, table_hbm, out_hbm, idx_v, buf0, buf1, g0, g1, s0, s1):
    wid = lax.axis_index("s") * NC + lax.axis_index("c")
    base = wid * BPW
    pltpu.sync_copy(x_hbm.at[pl.ds(base, BPW)], idx_v)

    bufs = (buf0, buf1)
    gsems = (g0, g1)
    ssems = (s0, s1)

    def gather(ch):
        b = ch % 2
        return pltpu.async_copy(
            table_hbm.at[idx_v.at[pl.ds(ch * CH, CH)]], bufs[b], gsems[b]
        )

    def store(ch):
        b = ch % 2
        return pltpu.async_copy(
            bufs[b], out_hbm.at[pl.ds(base + ch * CH, CH)], ssems[b]
        )

    g = [None] * NCH
    s = [None] * NCH
    g[0] = gather(0)
    g[1] = gather(1)
    for ch in range(NCH):
        g[ch].wait()
        s[ch] = store(ch)
        if ch + 2 < NCH:
            s[ch].wait()
            g[ch + 2] = gather(ch + 2)
    s[NCH - 2].wait()
    s[NCH - 1].wait()


def kernel(x, table):
    flat = _emb_lookup(x.reshape(-1), table)
    return flat.reshape(BATCH, FIELDS, EMBEDDING_DIM)
